# Initial kernel scaffold; baseline (speedup 1.0000x reference)
import jax
import jax.numpy as jnp
from jax.experimental import pallas as pl


def kernel(x, relative_position_bias_table):
    raise NotImplementedError("write your pallas kernel here")


# speedup vs baseline: 1.1415x; 1.1415x over previous
import functools

import jax
import jax.numpy as jnp
from jax import lax
from jax.experimental import pallas as pl
from jax.experimental.pallas import tpu as pltpu
from jax.experimental.pallas import tpu_sc as plsc

S = 256
MAX_LEN = 8000
ROW0 = MAX_LEN - 1 - (S - 1)
NROWS = 2 * S - 1
NC = 2
NS = 16
NW = NC * NS
HPW = S // NW
RB = 16
L = 16


def _sc_body(x_hbm, table_hbm, out_hbm, tblk, twin, xblk, sbuf):
    cid = lax.axis_index("c")
    sid = lax.axis_index("s")
    wid = sid * NC + cid
    h0 = wid * HPW

    pltpu.sync_copy(table_hbm.at[pl.ds(ROW0, NROWS), pl.ds(h0, HPW)],
                    tblk.at[pl.ds(0, NROWS), :])
    lane = lax.iota(jnp.int32, L)
    for hl in range(HPW):
        hsplat = jnp.full((L,), hl, jnp.int32)
        for cc in range(2 * S // L):
            rows = lane + cc * L
            twin[hl, pl.ds(cc * L, L)] = plsc.load_gather(tblk, [rows, hsplat])

    def iblock(ib_idx, carry):
        ib = ib_idx * RB
        pltpu.sync_copy(x_hbm.at[pl.ds(ib, RB), :], xblk)

        def irow(il, c2):
            base = (S - 1) - (ib + il)
            for cc in range(S // L):
                xv = xblk[il, pl.ds(cc * L, L)]
                for hl in range(HPW):
                    tv = twin[hl, pl.ds(base + cc * L, L)]
                    sbuf[hl, il, pl.ds(cc * L, L)] = xv + tv
            return c2

        lax.fori_loop(0, RB, irow, 0)
        for hl in range(HPW):
            pltpu.sync_copy(sbuf.at[hl],
                            out_hbm.at[h0 + hl, pl.ds(ib, RB), :])
        return carry

    lax.fori_loop(0, S // RB, iblock, 0)


@jax.jit
def _sc_call(xf, table):
    mesh = plsc.VectorSubcoreMesh(core_axis_name="c", subcore_axis_name="s")
    return pl.kernel(
        _sc_body,
        out_type=jax.ShapeDtypeStruct((S, S, S), jnp.float32),
        mesh=mesh,
        scratch_types=[
            pltpu.VMEM((NROWS + 1, HPW), jnp.float32),
            pltpu.VMEM((HPW, 2 * S), jnp.float32),
            pltpu.VMEM((RB, S), jnp.float32),
            pltpu.VMEM((HPW, RB, S), jnp.float32),
        ],
        compiler_params=pltpu.CompilerParams(use_tc_tiling_on_sc=False,
                                              needs_layout_passes=False),
        name="rel_pos_bias_sc",
    )(xf, table)


def kernel(x, relative_position_bias_table):
    xf = x[0]
    out = _sc_call(xf, relative_position_bias_table)
    return out[None]


# speedup vs baseline: 1.3273x; 1.1628x over previous
import functools

import jax
import jax.numpy as jnp
from jax import lax
from jax.experimental import pallas as pl
from jax.experimental.pallas import tpu as pltpu
from jax.experimental.pallas import tpu_sc as plsc

S = 256
MAX_LEN = 8000
ROW0 = MAX_LEN - 1 - (S - 1)
NROWS = 2 * S - 1
NC = 2
NS = 16
NW = NC * NS
HPW = S // NW
RB = 16
NB = S // RB
L = 16


def _sc_body(x_hbm, table_hbm, out_hbm, tblk, twin, xblk, sbuf, sem_x, sem_o):
    cid = lax.axis_index("c")
    sid = lax.axis_index("s")
    wid = sid * NC + cid
    h0 = wid * HPW

    pltpu.sync_copy(table_hbm.at[pl.ds(ROW0, NROWS), pl.ds(h0, HPW)],
                    tblk.at[pl.ds(0, NROWS), :])
    lane = lax.iota(jnp.int32, L)
    for hl in range(HPW):
        hsplat = jnp.full((L,), hl, jnp.int32)
        for cc in range(2 * S // L):
            rows = lane + cc * L
            twin[hl, pl.ds(cc * L, L)] = plsc.load_gather(tblk, [rows, hsplat])

    pltpu.async_copy(x_hbm.at[pl.ds(0, RB), :], xblk.at[0], sem_x.at[0])

    def iblock(k, carry):
        ib = k * RB
        p = lax.rem(k, 2)

        @pl.when(k >= 2)
        def _():
            for hl in range(HPW):
                pltpu.make_async_copy(
                    sbuf.at[p, hl],
                    out_hbm.at[h0 + hl, pl.ds(0, RB), :],
                    sem_o.at[p]).wait()

        pltpu.make_async_copy(x_hbm.at[pl.ds(ib, RB), :], xblk.at[p],
                              sem_x.at[p]).wait()

        @pl.when(k + 1 < NB)
        def _():
            pltpu.async_copy(x_hbm.at[pl.ds(ib + RB, RB), :],
                             xblk.at[1 - p], sem_x.at[1 - p])

        def irow(il, c2):
            base = (S - 1) - (ib + il)
            for cc in range(S // L):
                xv = xblk[p, il, pl.ds(cc * L, L)]
                for hl in range(HPW):
                    tv = twin[hl, pl.ds(base + cc * L, L)]
                    sbuf[p, hl, il, pl.ds(cc * L, L)] = xv + tv
            return c2

        lax.fori_loop(0, RB, irow, 0)

        for hl in range(HPW):
            pltpu.async_copy(sbuf.at[p, hl],
                             out_hbm.at[h0 + hl, pl.ds(ib, RB), :],
                             sem_o.at[p])
        return carry

    lax.fori_loop(0, NB, iblock, 0)

    for p in range(2):
        for hl in range(HPW):
            pltpu.make_async_copy(
                sbuf.at[p, hl],
                out_hbm.at[h0 + hl, pl.ds(0, RB), :],
                sem_o.at[p]).wait()


@jax.jit
def _sc_call(xf, table):
    mesh = plsc.VectorSubcoreMesh(core_axis_name="c", subcore_axis_name="s")
    return pl.kernel(
        _sc_body,
        out_type=jax.ShapeDtypeStruct((S, S, S), jnp.float32),
        mesh=mesh,
        scratch_types=[
            pltpu.VMEM((NROWS + 1, HPW), jnp.float32),
            pltpu.VMEM((HPW, 2 * S), jnp.float32),
            pltpu.VMEM((2, RB, S), jnp.float32),
            pltpu.VMEM((2, HPW, RB, S), jnp.float32),
            pltpu.SemaphoreType.DMA((2,)),
            pltpu.SemaphoreType.DMA((2,)),
        ],
        compiler_params=pltpu.CompilerParams(use_tc_tiling_on_sc=False,
                                             needs_layout_passes=False),
        name="rel_pos_bias_sc",
    )(xf, table)


def kernel(x, relative_position_bias_table):
    xf = x[0]
    out = _sc_call(xf, relative_position_bias_table)
    return out[None]


# speedup vs baseline: 2.2842x; 1.7209x over previous
import functools

import jax
import jax.numpy as jnp
from jax import lax
from jax.experimental import pallas as pl
from jax.experimental.pallas import tpu as pltpu
from jax.experimental.pallas import tpu_sc as plsc

S = 256
MAX_LEN = 8000
ROW0 = MAX_LEN - 1 - (S - 1)
NROWS = 2 * S - 1
NC = 2
NS = 16
NW = NC * NS
HPW = S // NW
RB = 16
NB = S // RB
L = 16


def _sc_body(x_hbm, table_hbm, out_hbm, tblk, twin, xblk, sbuf, sem_x, sem_o):
    cid = lax.axis_index("c")
    sid = lax.axis_index("s")
    wid = sid * NC + cid
    h0 = wid * HPW

    pltpu.sync_copy(table_hbm.at[pl.ds(ROW0, NROWS), pl.ds(h0, HPW)],
                    tblk.at[pl.ds(0, NROWS), :])
    lane = lax.iota(jnp.int32, L)
    for hl in range(HPW):
        hsplat = jnp.full((L,), hl, jnp.int32)
        for cc in range(2 * S // L):
            rows = lane + cc * L
            twin[hl, pl.ds(cc * L, L)] = plsc.load_gather(tblk, [rows, hsplat])

    pltpu.async_copy(x_hbm.at[pl.ds(0, RB), :], xblk.at[0], sem_x.at[0])

    def iblock(k, carry):
        ib = k * RB
        p = lax.rem(k, 2)

        @pl.when(k >= 2)
        def _():
            for hl in range(HPW):
                pltpu.make_async_copy(
                    sbuf.at[p, hl],
                    out_hbm.at[h0 + hl, pl.ds(0, RB), :],
                    sem_o.at[p]).wait()

        pltpu.make_async_copy(x_hbm.at[pl.ds(ib, RB), :], xblk.at[p],
                              sem_x.at[p]).wait()

        @pl.when(k + 1 < NB)
        def _():
            pltpu.async_copy(x_hbm.at[pl.ds(ib + RB, RB), :],
                             xblk.at[1 - p], sem_x.at[1 - p])

        @plsc.parallel_loop(0, RB * (S // L), unroll=8)
        def _(t):
            il = lax.shift_right_logical(t, 4)
            off = lax.shift_left(lax.bitwise_and(t, S // L - 1), 4)
            base = (S - 1) - (ib + il)
            xv = xblk[p, il, pl.ds(off, L)]
            for hl in range(HPW):
                tv = twin[hl, pl.ds(base + off, L)]
                sbuf[p, hl, il, pl.ds(off, L)] = xv + tv

        for hl in range(HPW):
            pltpu.async_copy(sbuf.at[p, hl],
                             out_hbm.at[h0 + hl, pl.ds(ib, RB), :],
                             sem_o.at[p])
        return carry

    lax.fori_loop(0, NB, iblock, 0)

    for p in range(2):
        for hl in range(HPW):
            pltpu.make_async_copy(
                sbuf.at[p, hl],
                out_hbm.at[h0 + hl, pl.ds(0, RB), :],
                sem_o.at[p]).wait()


@jax.jit
def _sc_call(xf, table):
    mesh = plsc.VectorSubcoreMesh(core_axis_name="c", subcore_axis_name="s")
    return pl.kernel(
        _sc_body,
        out_type=jax.ShapeDtypeStruct((S, S, S), jnp.float32),
        mesh=mesh,
        scratch_types=[
            pltpu.VMEM((NROWS + 1, HPW), jnp.float32),
            pltpu.VMEM((HPW, 2 * S), jnp.float32),
            pltpu.VMEM((2, RB, S), jnp.float32),
            pltpu.VMEM((2, HPW, RB, S), jnp.float32),
            pltpu.SemaphoreType.DMA((2,)),
            pltpu.SemaphoreType.DMA((2,)),
        ],
        compiler_params=pltpu.CompilerParams(use_tc_tiling_on_sc=False,
                                             needs_layout_passes=False),
        name="rel_pos_bias_sc",
    )(xf, table)


def kernel(x, relative_position_bias_table):
    xf = x[0]
    out = _sc_call(xf, relative_position_bias_table)
    return out[None]


# speedup vs baseline: 4.0066x; 1.7540x over previous
import functools

import jax
import jax.numpy as jnp
from jax import lax
from jax.experimental import pallas as pl
from jax.experimental.pallas import tpu as pltpu
from jax.experimental.pallas import tpu_sc as plsc

S = 256
MAX_LEN = 8000
ROW0 = MAX_LEN - 1 - (S - 1)
NROWS = 2 * S - 1
NC = 2
NS = 16
NW = NC * NS
HPW = S // NW
RB = 8
NB = S // RB
L = 16


def _sc_body(x_hbm, table_hbm, out_hbm, tblk, twin, xblk, sbuf, sem_x, sem_o):
    cid = lax.axis_index("c")
    sid = lax.axis_index("s")
    wid = sid * NC + cid
    h0 = wid * HPW

    ct = lax.div(h0, 128) * 128
    hcol = lax.rem(h0, 128)
    lane = lax.iota(jnp.int32, L)
    pltpu.sync_copy(table_hbm.at[pl.ds(ROW0, 2 * S), pl.ds(ct, 128)], tblk)
    for hl in range(HPW):
        hsplat = jnp.full((L,), 0, jnp.int32) + (hcol + hl)
        for cc in range(2 * S // L):
            rows = lane + cc * L
            twin[hl, pl.ds(cc * L, L)] = plsc.load_gather(tblk, [rows, hsplat])

    pltpu.async_copy(x_hbm.at[pl.ds(0, RB), :], xblk.at[0], sem_x.at[0])

    def iblock(k, carry):
        ib = k * RB
        p = lax.rem(k, 2)

        @pl.when(k >= 2)
        def _():
            for hl in range(HPW):
                pltpu.make_async_copy(
                    sbuf.at[p, hl],
                    out_hbm.at[h0 + hl, pl.ds(0, RB), :],
                    sem_o.at[p]).wait()

        pltpu.make_async_copy(x_hbm.at[pl.ds(ib, RB), :], xblk.at[p],
                              sem_x.at[p]).wait()

        @pl.when(k + 1 < NB)
        def _():
            pltpu.async_copy(x_hbm.at[pl.ds(ib + RB, RB), :],
                             xblk.at[1 - p], sem_x.at[1 - p])

        lane2 = lax.iota(jnp.int32, L)

        @plsc.parallel_loop(0, RB * (S // L), unroll=8)
        def _(t):
            il = lax.shift_right_logical(t, 4)
            off = lax.shift_left(lax.bitwise_and(t, S // L - 1), 4)
            base = (S - 1) - (ib + il)
            rows = base + off + lane2
            xv = xblk[p, il, pl.ds(off, L)]
            for hl in range(HPW):
                tv = plsc.load_gather(
                    twin, [jnp.full((L,), 0, jnp.int32) + hl, rows])
                sbuf[p, hl, il, pl.ds(off, L)] = xv + tv

        for hl in range(HPW):
            pltpu.async_copy(sbuf.at[p, hl],
                             out_hbm.at[h0 + hl, pl.ds(ib, RB), :],
                             sem_o.at[p])
        return carry

    lax.fori_loop(0, NB, iblock, 0)

    for p in range(2):
        for hl in range(HPW):
            pltpu.make_async_copy(
                sbuf.at[p, hl],
                out_hbm.at[h0 + hl, pl.ds(0, RB), :],
                sem_o.at[p]).wait()


@jax.jit
def _sc_call(xf, table):
    mesh = plsc.VectorSubcoreMesh(core_axis_name="c", subcore_axis_name="s")
    return pl.kernel(
        _sc_body,
        out_type=jax.ShapeDtypeStruct((S, S, S), jnp.float32),
        mesh=mesh,
        scratch_types=[
            pltpu.VMEM((2 * S, 128), jnp.float32),
            pltpu.VMEM((HPW, 2 * S), jnp.float32),
            pltpu.VMEM((2, RB, S), jnp.float32),
            pltpu.VMEM((2, HPW, RB, S), jnp.float32),
            pltpu.SemaphoreType.DMA((2,)),
            pltpu.SemaphoreType.DMA((2,)),
        ],
        compiler_params=pltpu.CompilerParams(use_tc_tiling_on_sc=True,
                                             needs_layout_passes=False),
        name="rel_pos_bias_sc",
    )(xf, table)


def kernel(x, relative_position_bias_table):
    xf = x[0]
    out = _sc_call(xf, relative_position_bias_table)
    return out[None]
